# Initial kernel scaffold; baseline (speedup 1.0000x reference)
import functools

import jax
import jax.numpy as jnp
from jax import lax
from jax.experimental import pallas as pl
from jax.experimental.pallas import tpu as pltpu
from jax.experimental.pallas import tpu_sc as plsc

NC = 2
NS = 16
LANES = 16
NW = NC * NS
CH = 128


def _sc_aggregate(x, src3, dst3, zeros_f, zeros_d, ones_d, r_pad, k_chunks):
    n, f = x.shape
    rps = r_pad // NS

    mesh = plsc.VectorSubcoreMesh(core_axis_name="c", subcore_axis_name="s")

    @functools.partial(
        pl.kernel,
        out_type=(
            jax.ShapeDtypeStruct((NC * r_pad, f), jnp.float32),
            jax.ShapeDtypeStruct((NC * r_pad, LANES), jnp.float32),
        ),
        mesh=mesh,
        scratch_types=[
            pltpu.VMEM((k_chunks, CH), jnp.int32),
            pltpu.VMEM((k_chunks, CH), jnp.int32),
            pltpu.VMEM((CH, f), jnp.float32),
            pltpu.VMEM((CH, LANES), jnp.float32),
            pltpu.VMEM_SHARED((r_pad, f), jnp.float32),
            pltpu.VMEM_SHARED((r_pad, LANES), jnp.float32),
        ],
    )
    def sc_kernel(x_hbm, src_hbm, dst_hbm, zf_hbm, zd_hbm, ones_hbm,
                  agg_out, deg_out,
                  src_v, dst_v, rows_v, ones_v, agg_sh, deg_sh):
        c = lax.axis_index("c")
        s = lax.axis_index("s")
        wid = s * NC + c
        base = s * rps

        pltpu.sync_copy(zf_hbm, agg_sh.at[pl.ds(base, rps)])
        pltpu.sync_copy(zd_hbm, deg_sh.at[pl.ds(base, rps)])
        pltpu.sync_copy(ones_hbm, ones_v)
        pltpu.sync_copy(src_hbm.at[wid], src_v)
        pltpu.sync_copy(dst_hbm.at[wid], dst_v)
        plsc.subcore_barrier()

        @pl.loop(0, k_chunks)
        def _(j):
            pltpu.sync_copy(x_hbm.at[src_v.at[j]], rows_v)
            pltpu.sync_copy(rows_v, agg_sh.at[dst_v.at[j]], add=True)
            pltpu.sync_copy(ones_v, deg_sh.at[dst_v.at[j]], add=True)

        plsc.subcore_barrier()
        out_base = c * r_pad + base
        pltpu.sync_copy(agg_sh.at[pl.ds(base, rps)],
                        agg_out.at[pl.ds(out_base, rps)])
        pltpu.sync_copy(deg_sh.at[pl.ds(base, rps)],
                        deg_out.at[pl.ds(out_base, rps)])

    return sc_kernel(x, src3, dst3, zeros_f, zeros_d, ones_d)


def _tc_finish(x, agg_a, agg_b, deg_a, deg_b, W, b):
    n, f = x.shape
    o = W.shape[0]
    br = 2000

    def body(x_ref, aa_ref, ab_ref, da_ref, db_ref, w_ref, b_ref, o_ref):
        agg = aa_ref[...] + ab_ref[...]
        deg = da_ref[...][:, :1] + db_ref[...][:, :1]
        h_n = agg / jnp.maximum(deg, 1.0)
        w1 = w_ref[:, :f]
        w2 = w_ref[:, f:]
        acc = lax.dot_general(x_ref[...], w1, (((1,), (1,)), ((), ())),
                              preferred_element_type=jnp.float32)
        acc = acc + lax.dot_general(h_n, w2, (((1,), (1,)), ((), ())),
                                    preferred_element_type=jnp.float32)
        o_ref[...] = acc + b_ref[...]

    return pl.pallas_call(
        body,
        grid=(n // br,),
        in_specs=[
            pl.BlockSpec((br, f), lambda i: (i, 0)),
            pl.BlockSpec((br, f), lambda i: (i, 0)),
            pl.BlockSpec((br, f), lambda i: (i, 0)),
            pl.BlockSpec((br, LANES), lambda i: (i, 0)),
            pl.BlockSpec((br, LANES), lambda i: (i, 0)),
            pl.BlockSpec((f, 2 * f), lambda i: (0, 0)),
            pl.BlockSpec((1, o), lambda i: (0, 0)),
        ],
        out_specs=pl.BlockSpec((br, o), lambda i: (i, 0)),
        out_shape=jax.ShapeDtypeStruct((n, o), jnp.float32),
    )(x, agg_a, agg_b, deg_a, deg_b, W, b.reshape(1, o))


def kernel(x, edge_index, W, b):
    n, f = x.shape
    e = edge_index.shape[1]

    k_chunks = -(-e // (NW * CH))
    e_pad = k_chunks * NW * CH
    r_pad = -(-(n + 1) // (NS * 8)) * (NS * 8)

    src = edge_index[0].astype(jnp.int32)
    dst = edge_index[1].astype(jnp.int32)
    pad = e_pad - e
    src = jnp.concatenate([src, jnp.zeros((pad,), jnp.int32)])
    dst = jnp.concatenate([dst, jnp.full((pad,), n, jnp.int32)])
    src3 = src.reshape(NW, k_chunks, CH)
    dst3 = dst.reshape(NW, k_chunks, CH)

    rps = r_pad // NS
    zeros_f = jnp.zeros((rps, f), jnp.float32)
    zeros_d = jnp.zeros((rps, LANES), jnp.float32)
    ones_d = jnp.ones((CH, LANES), jnp.float32)

    agg, deg = _sc_aggregate(x, src3, dst3, zeros_f, zeros_d, ones_d,
                             r_pad, k_chunks)
    agg_a = agg[:n]
    agg_b = agg[r_pad:r_pad + n]
    deg_a = deg[:n]
    deg_b = deg[r_pad:r_pad + n]
    return _tc_finish(x, agg_a, agg_b, deg_a, deg_b, W, b)


# speedup vs baseline: 3.6221x; 3.6221x over previous
import functools

import jax
import jax.numpy as jnp
from jax import lax
from jax.experimental import pallas as pl
from jax.experimental.pallas import tpu as pltpu
from jax.experimental.pallas import tpu_sc as plsc

NC = 2
NS = 16
NW = NC * NS
CH = 64


def _sc_aggregate(x, src2, dst2, zeros_f, ones_f, r_pad, n_chunks):
    n, f = x.shape
    rps = r_pad // NS

    mesh = plsc.VectorSubcoreMesh(core_axis_name="c", subcore_axis_name="s")

    @functools.partial(
        pl.kernel,
        out_type=(
            jax.ShapeDtypeStruct((NC * r_pad, f), jnp.float32),
            jax.ShapeDtypeStruct((NC * r_pad, f), jnp.float32),
        ),
        mesh=mesh,
        scratch_types=[
            pltpu.VMEM((CH,), jnp.int32),
            pltpu.VMEM((CH,), jnp.int32),
            pltpu.VMEM((CH, f), jnp.float32),
            pltpu.VMEM((CH, f), jnp.float32),
            pltpu.VMEM_SHARED((r_pad, f), jnp.float32),
            pltpu.SemaphoreType.DMA,
        ],
    )
    def sc_kernel(x_hbm, src_hbm, dst_hbm, zf_hbm, ones_hbm,
                  agg_out, deg_out,
                  src_v, dst_v, rows_v, ones_v, sh, sem):
        c = lax.axis_index("c")
        s = lax.axis_index("s")
        wid = s * NC + c
        base = s * rps
        out_base = c * r_pad + base

        pltpu.sync_copy(zf_hbm, sh.at[pl.ds(base, rps)])
        pltpu.sync_copy(ones_hbm, ones_v)
        plsc.subcore_barrier()

        @pl.loop(0, n_chunks)
        def _(j):
            row = wid * n_chunks + j
            pltpu.sync_copy(src_hbm.at[row], src_v)
            pltpu.sync_copy(dst_hbm.at[row], dst_v)
            pltpu.async_copy(x_hbm.at[src_v], rows_v, sem).wait()
            pltpu.sync_copy(rows_v, sh.at[dst_v], add=True)

        plsc.subcore_barrier()
        pltpu.sync_copy(sh.at[pl.ds(base, rps)],
                        agg_out.at[pl.ds(out_base, rps)])
        plsc.subcore_barrier()

        pltpu.sync_copy(zf_hbm, sh.at[pl.ds(base, rps)])
        plsc.subcore_barrier()

        @pl.loop(0, n_chunks)
        def _(j):
            row = wid * n_chunks + j
            pltpu.sync_copy(dst_hbm.at[row], dst_v)
            pltpu.sync_copy(ones_v, sh.at[dst_v], add=True)

        plsc.subcore_barrier()
        pltpu.sync_copy(sh.at[pl.ds(base, rps)],
                        deg_out.at[pl.ds(out_base, rps)])

    return sc_kernel(x, src2, dst2, zeros_f, ones_f)


def _tc_finish(x, agg_a, agg_b, deg_a, deg_b, W, b):
    n, f = x.shape
    o = W.shape[0]
    br = 2000

    def body(x_ref, aa_ref, ab_ref, da_ref, db_ref, w_ref, b_ref, o_ref):
        agg = aa_ref[...] + ab_ref[...]
        deg_col = da_ref[:, :1] + db_ref[:, :1]
        h_n = agg / jnp.maximum(deg_col, 1.0)
        w1 = w_ref[:, :f]
        w2 = w_ref[:, f:]
        acc = lax.dot_general(x_ref[...], w1, (((1,), (1,)), ((), ())),
                              preferred_element_type=jnp.float32)
        acc = acc + lax.dot_general(h_n, w2, (((1,), (1,)), ((), ())),
                                    preferred_element_type=jnp.float32)
        o_ref[...] = acc + b_ref[...]

    return pl.pallas_call(
        body,
        grid=(n // br,),
        in_specs=[
            pl.BlockSpec((br, f), lambda i: (i, 0)),
            pl.BlockSpec((br, f), lambda i: (i, 0)),
            pl.BlockSpec((br, f), lambda i: (i, 0)),
            pl.BlockSpec((br, f), lambda i: (i, 0)),
            pl.BlockSpec((br, f), lambda i: (i, 0)),
            pl.BlockSpec((f, 2 * f), lambda i: (0, 0)),
            pl.BlockSpec((1, o), lambda i: (0, 0)),
        ],
        out_specs=pl.BlockSpec((br, o), lambda i: (i, 0)),
        out_shape=jax.ShapeDtypeStruct((n, o), jnp.float32),
    )(x, agg_a, agg_b, deg_a, deg_b, W, b.reshape(1, o))


def kernel(x, edge_index, W, b):
    n, f = x.shape
    e = edge_index.shape[1]

    n_chunks = -(-e // (NW * CH))
    e_pad = n_chunks * NW * CH
    r_pad = -(-(n + 1) // (NS * 8)) * (NS * 8)

    src = edge_index[0].astype(jnp.int32)
    dst = edge_index[1].astype(jnp.int32)
    pad = e_pad - e
    src = jnp.concatenate([src, jnp.zeros((pad,), jnp.int32)])
    dst = jnp.concatenate([dst, jnp.full((pad,), n, jnp.int32)])
    src2 = src.reshape(NW * n_chunks, CH)
    dst2 = dst.reshape(NW * n_chunks, CH)

    rps = r_pad // NS
    zeros_f = jnp.zeros((rps, f), jnp.float32)
    ones_f = jnp.ones((CH, f), jnp.float32)

    agg, deg = _sc_aggregate(x, src2, dst2, zeros_f, ones_f,
                             r_pad, n_chunks)
    agg_a = agg[:n]
    agg_b = agg[r_pad:r_pad + n]
    deg_a = deg[:n]
    deg_b = deg[r_pad:r_pad + n]
    return _tc_finish(x, agg_a, agg_b, deg_a, deg_b, W, b)
